# baseline (speedup 1.0000x reference)
import jax
import jax.numpy as jnp
from jax import lax
from jax.experimental import pallas as pl
from jax.experimental.pallas import tpu as pltpu
from jax.experimental.pallas import tpu_sc as plsc

N_NODES = 10000
N_EDGES = 320000
D_FEAT = 128
GCN_OUT = 100
HP = 112
ONES_COL = 100
NUM_CORES = 2
NUM_SUBCORES = 16
NW = NUM_CORES * NUM_SUBCORES
EDGES_PER_WORKER = N_EDGES // NW
CHUNK = 100
NCHUNK = EDGES_PER_WORKER // CHUNK
ZROWS = 80
NBUF = 3
N_PAD = 10240
ROWS_PER_TILE = N_PAD // NUM_SUBCORES
ROW_BLK = 2000


def _pre_body(x_ref, w_ref, c_ref, y_ref):
    y_ref[...] = (
        jnp.dot(x_ref[...], w_ref[...], preferred_element_type=jnp.float32)
        + c_ref[...]
    )


def _post_body(a_ref, b_ref, b1_ref, w2_ref, b2_ref, o_ref):
    s = a_ref[0] + b_ref[0]
    deg = jnp.maximum(s[:, ONES_COL:ONES_COL + 1], 1.0)
    h = jnp.maximum(s[:, :GCN_OUT] / deg + b1_ref[...], 0.0)
    o_ref[...] = (
        jnp.dot(h, w2_ref[...], preferred_element_type=jnp.float32)
        + b2_ref[...]
    )


def _sc_agg_body(y_hbm, idx_hbm, out_hbm,
                 idxv, rows_0, rows_1, rows_2, agg,
                 gs_0, gs_1, gs_2):
    bufs = (rows_0, rows_1, rows_2)
    sems = (gs_0, gs_1, gs_2)
    c = lax.axis_index("c")
    s = lax.axis_index("s")
    wid = c * NUM_SUBCORES + s
    r0 = s * ROWS_PER_TILE

    ds_ = pltpu.async_copy(idx_hbm.at[0, wid], idxv.at[0], gs_0)
    dd_ = pltpu.async_copy(idx_hbm.at[1, wid], idxv.at[1], gs_1)

    z = jnp.zeros((16,), jnp.float32)

    def zrow(i, carry):
        for k in range(HP // 16):
            rows_0[i, pl.ds(k * 16, 16)] = z
        return carry

    lax.fori_loop(0, ZROWS, zrow, 0)
    for t in range(ROWS_PER_TILE // ZROWS):
        pltpu.sync_copy(rows_0.at[pl.ds(0, ZROWS)],
                        agg.at[pl.ds(r0 + t * ZROWS, ZROWS)])
    plsc.subcore_barrier()
    ds_.wait()
    dd_.wait()

    last = NCHUNK - 1
    for b in range(NBUF):
        pltpu.async_copy(y_hbm.at[idxv.at[0, b]], bufs[b], sems[b])

    def chunk(i, carry):
        for b in range(NBUF):
            j = NBUF * i + b
            pltpu.make_async_copy(y_hbm.at[idxv.at[0, j]], bufs[b],
                                  sems[b]).wait()
            pltpu.sync_copy(bufs[b], agg.at[idxv.at[1, j]], add=True)
            nj = jnp.minimum(j + NBUF, last)
            pltpu.async_copy(y_hbm.at[idxv.at[0, nj]], bufs[b], sems[b])
        return carry

    lax.fori_loop(0, NCHUNK // NBUF, chunk, 0)

    for j in range(NCHUNK - NCHUNK % NBUF, NCHUNK):
        b = j % NBUF
        pltpu.make_async_copy(y_hbm.at[idxv.at[0, j]], bufs[b],
                              sems[b]).wait()
        pltpu.sync_copy(bufs[b], agg.at[idxv.at[1, j]], add=True)
    for b in range(NCHUNK % NBUF, NBUF):
        pltpu.make_async_copy(y_hbm.at[idxv.at[0, last]], bufs[b],
                              sems[b]).wait()
    plsc.subcore_barrier()

    pltpu.sync_copy(
        agg.at[pl.ds(r0, ROWS_PER_TILE)],
        out_hbm.at[c, pl.ds(r0, ROWS_PER_TILE)],
    )


_sc_agg = pl.kernel(
    _sc_agg_body,
    out_type=jax.ShapeDtypeStruct((NUM_CORES, N_PAD, HP), jnp.float32),
    mesh=plsc.VectorSubcoreMesh(
        core_axis_name="c", subcore_axis_name="s",
        num_cores=NUM_CORES, num_subcores=NUM_SUBCORES,
    ),
    compiler_params=pltpu.CompilerParams(use_tc_tiling_on_sc=False),
    scratch_types=[
        pltpu.VMEM((2, NCHUNK, CHUNK), jnp.int32),
        pltpu.VMEM((CHUNK, HP), jnp.float32),
        pltpu.VMEM((CHUNK, HP), jnp.float32),
        pltpu.VMEM((CHUNK, HP), jnp.float32),
        pltpu.VMEM_SHARED((N_PAD, HP), jnp.float32),
        pltpu.SemaphoreType.DMA,
        pltpu.SemaphoreType.DMA,
        pltpu.SemaphoreType.DMA,
    ],
)


def kernel(x, edge_index, W1, b1, W2, b2):
    idx = edge_index.reshape(2, NW, NCHUNK, CHUNK)
    W1p = jnp.pad(W1, ((0, 0), (0, HP - GCN_OUT)))
    cvec = jnp.zeros((1, HP), jnp.float32).at[0, ONES_COL].set(1.0)

    y = pl.pallas_call(
        _pre_body,
        grid=(N_NODES // ROW_BLK,),
        in_specs=[
            pl.BlockSpec((ROW_BLK, D_FEAT), lambda i: (i, 0)),
            pl.BlockSpec((D_FEAT, HP), lambda i: (0, 0)),
            pl.BlockSpec((1, HP), lambda i: (0, 0)),
        ],
        out_specs=pl.BlockSpec((ROW_BLK, HP), lambda i: (i, 0)),
        out_shape=jax.ShapeDtypeStruct((N_NODES, HP), jnp.float32),
    )(x, W1p, cvec)

    part = _sc_agg(y, idx)

    logits = pl.pallas_call(
        _post_body,
        grid=(N_NODES // ROW_BLK,),
        in_specs=[
            pl.BlockSpec((1, ROW_BLK, HP), lambda i: (0, i, 0)),
            pl.BlockSpec((1, ROW_BLK, HP), lambda i: (1, i, 0)),
            pl.BlockSpec((1, GCN_OUT), lambda i: (0, 0)),
            pl.BlockSpec((GCN_OUT, 1), lambda i: (0, 0)),
            pl.BlockSpec((1, 1), lambda i: (0, 0)),
        ],
        out_specs=pl.BlockSpec((ROW_BLK, 1), lambda i: (i, 0)),
        out_shape=jax.ShapeDtypeStruct((N_NODES, 1), jnp.float32),
    )(part, part, b1.reshape(1, GCN_OUT), W2, b2.reshape(1, 1))
    return logits


# speedup vs baseline: 1.0589x; 1.0589x over previous
import jax
import jax.numpy as jnp
from jax import lax
from jax.experimental import pallas as pl
from jax.experimental.pallas import tpu as pltpu
from jax.experimental.pallas import tpu_sc as plsc

N_NODES = 10000
N_EDGES = 320000
D_FEAT = 128
GCN_OUT = 100
HP = 112
ONES_COL = 100
NUM_CORES = 2
NUM_SUBCORES = 16
NW = NUM_CORES * NUM_SUBCORES
EDGES_PER_WORKER = N_EDGES // NW
CHUNK = 80
NCHUNK = EDGES_PER_WORKER // CHUNK
NBUF = 4
N_PAD = 10240
ROWS_PER_TILE = N_PAD // NUM_SUBCORES
ROW_BLK = 2000


def _pre_body(x_ref, w_ref, c_ref, y_ref):
    y_ref[...] = (
        jnp.dot(x_ref[...], w_ref[...], preferred_element_type=jnp.float32)
        + c_ref[...]
    )


def _post_body(a_ref, b_ref, b1_ref, w2_ref, b2_ref, o_ref):
    s = a_ref[0] + b_ref[0]
    deg = jnp.maximum(s[:, ONES_COL:ONES_COL + 1], 1.0)
    h = jnp.maximum(s[:, :GCN_OUT] / deg + b1_ref[...], 0.0)
    o_ref[...] = (
        jnp.dot(h, w2_ref[...], preferred_element_type=jnp.float32)
        + b2_ref[...]
    )


def _sc_agg_body(y_hbm, idx_hbm, out_hbm,
                 idxv, rows_0, rows_1, rows_2, rows_3, agg,
                 gs_0, gs_1, gs_2, gs_3):
    bufs = (rows_0, rows_1, rows_2, rows_3)
    sems = (gs_0, gs_1, gs_2, gs_3)
    c = lax.axis_index("c")
    s = lax.axis_index("s")
    wid = c * NUM_SUBCORES + s
    r0 = s * ROWS_PER_TILE

    ds_ = pltpu.async_copy(idx_hbm.at[0, wid], idxv.at[0], gs_0)
    dd_ = pltpu.async_copy(idx_hbm.at[1, wid], idxv.at[1], gs_1)

    z = jnp.zeros((16,), jnp.float32)

    def zrow(i, carry):
        for k in range(HP // 16):
            rows_0[i, pl.ds(k * 16, 16)] = z
        return carry

    lax.fori_loop(0, CHUNK, zrow, 0)
    for t in range(ROWS_PER_TILE // CHUNK):
        pltpu.sync_copy(rows_0, agg.at[pl.ds(r0 + t * CHUNK, CHUNK)])

    ds_.wait()
    dd_.wait()
    last = NCHUNK - 1
    for b in range(NBUF):
        pltpu.async_copy(y_hbm.at[idxv.at[0, b]], bufs[b], sems[b])
    plsc.subcore_barrier()


    def chunk(i, carry):
        for b in range(NBUF):
            j = NBUF * i + b
            pltpu.make_async_copy(y_hbm.at[idxv.at[0, j]], bufs[b],
                                  sems[b]).wait()
            pltpu.sync_copy(bufs[b], agg.at[idxv.at[1, j]], add=True)
            nj = jnp.minimum(j + NBUF, last)
            pltpu.async_copy(y_hbm.at[idxv.at[0, nj]], bufs[b], sems[b])
        return carry

    lax.fori_loop(0, NCHUNK // NBUF, chunk, 0)

    for j in range(NCHUNK - NCHUNK % NBUF, NCHUNK):
        b = j % NBUF
        pltpu.make_async_copy(y_hbm.at[idxv.at[0, j]], bufs[b],
                              sems[b]).wait()
        pltpu.sync_copy(bufs[b], agg.at[idxv.at[1, j]], add=True)
    for b in range(NCHUNK % NBUF, NBUF):
        pltpu.make_async_copy(y_hbm.at[idxv.at[0, last]], bufs[b],
                              sems[b]).wait()
    plsc.subcore_barrier()

    pltpu.sync_copy(
        agg.at[pl.ds(r0, ROWS_PER_TILE)],
        out_hbm.at[c, pl.ds(r0, ROWS_PER_TILE)],
    )


_sc_agg = pl.kernel(
    _sc_agg_body,
    out_type=jax.ShapeDtypeStruct((NUM_CORES, N_PAD, HP), jnp.float32),
    mesh=plsc.VectorSubcoreMesh(
        core_axis_name="c", subcore_axis_name="s",
        num_cores=NUM_CORES, num_subcores=NUM_SUBCORES,
    ),
    compiler_params=pltpu.CompilerParams(use_tc_tiling_on_sc=False),
    scratch_types=[
        pltpu.VMEM((2, NCHUNK, CHUNK), jnp.int32),
        pltpu.VMEM((CHUNK, HP), jnp.float32),
        pltpu.VMEM((CHUNK, HP), jnp.float32),
        pltpu.VMEM((CHUNK, HP), jnp.float32),
        pltpu.VMEM((CHUNK, HP), jnp.float32),
        pltpu.VMEM_SHARED((N_PAD, HP), jnp.float32),
        pltpu.SemaphoreType.DMA,
        pltpu.SemaphoreType.DMA,
        pltpu.SemaphoreType.DMA,
        pltpu.SemaphoreType.DMA,
    ],
)


def kernel(x, edge_index, W1, b1, W2, b2):
    idx = edge_index.reshape(2, NW, NCHUNK, CHUNK)
    W1p = jnp.pad(W1, ((0, 0), (0, HP - GCN_OUT)))
    cvec = jnp.zeros((1, HP), jnp.float32).at[0, ONES_COL].set(1.0)

    y = pl.pallas_call(
        _pre_body,
        grid=(N_NODES // ROW_BLK,),
        in_specs=[
            pl.BlockSpec((ROW_BLK, D_FEAT), lambda i: (i, 0)),
            pl.BlockSpec((D_FEAT, HP), lambda i: (0, 0)),
            pl.BlockSpec((1, HP), lambda i: (0, 0)),
        ],
        out_specs=pl.BlockSpec((ROW_BLK, HP), lambda i: (i, 0)),
        out_shape=jax.ShapeDtypeStruct((N_NODES, HP), jnp.float32),
    )(x, W1p, cvec)

    part = _sc_agg(y, idx)

    logits = pl.pallas_call(
        _post_body,
        grid=(N_NODES // ROW_BLK,),
        in_specs=[
            pl.BlockSpec((1, ROW_BLK, HP), lambda i: (0, i, 0)),
            pl.BlockSpec((1, ROW_BLK, HP), lambda i: (1, i, 0)),
            pl.BlockSpec((1, GCN_OUT), lambda i: (0, 0)),
            pl.BlockSpec((GCN_OUT, 1), lambda i: (0, 0)),
            pl.BlockSpec((1, 1), lambda i: (0, 0)),
        ],
        out_specs=pl.BlockSpec((ROW_BLK, 1), lambda i: (i, 0)),
        out_shape=jax.ShapeDtypeStruct((N_NODES, 1), jnp.float32),
    )(part, part, b1.reshape(1, GCN_OUT), W2, b2.reshape(1, 1))
    return logits
